# baseline (speedup 1.0000x reference)
import functools

import jax
import jax.numpy as jnp
from jax.experimental import pallas as pl
from jax.experimental.pallas import tpu as pltpu

_L = (
    (32, 3, 64, False),
    (32, 64, 64, True),
    (16, 64, 128, False),
    (16, 128, 128, True),
    (8, 128, 256, False),
    (8, 256, 256, False),
    (8, 256, 256, False),
    (8, 256, 256, True),
    (4, 256, 512, False),
)


def _fill_borders(P, H, W):
    P[:, pl.ds(0, 1), pl.ds(1, W), :] = P[:, pl.ds(2, 1), pl.ds(1, W), :]
    P[:, pl.ds(H + 1, 1), pl.ds(1, W), :] = P[:, pl.ds(H - 1, 1), pl.ds(1, W), :]
    P[:, :, pl.ds(0, 1), :] = P[:, :, pl.ds(2, 1), :]
    P[:, :, pl.ds(W + 1, 1), :] = P[:, :, pl.ds(W - 1, 1), :]


def _layer(li, P, w_ref, b_ref, BBh):
    H, cin, cout, pool = _L[li]
    W = H
    R = BBh * (H + 2) * W
    patch = jnp.concatenate(
        [P[:, :, dx:dx + W, 0:cin].reshape(R, cin) for dx in range(3)],
        axis=1)
    y = jnp.dot(patch, w_ref[...], preferred_element_type=jnp.float32)
    y = y.reshape(BBh, H + 2, W, 3 * cout)
    acc = (y[:, 0:H, :, 0:cout]
           + y[:, 1:H + 1, :, cout:2 * cout]
           + y[:, 2:H + 2, :, 2 * cout:3 * cout])
    if pool:
        Ho = H // 2
        a5 = acc.reshape(BBh, Ho, 2, W, cout)
        a = jnp.maximum(a5[:, :, 0], a5[:, :, 1])
        b5 = a.reshape(BBh, Ho, Ho, 2, cout)
        acc = jnp.maximum(b5[:, :, :, 0], b5[:, :, :, 1])
    else:
        Ho = H
    r = jnp.maximum(acc + b_ref[...], 0.0)
    r = r.astype(jnp.bfloat16)
    if li < 8:
        rb = jnp.concatenate([r[:, 1:2], r, r[:, Ho - 2:Ho - 1]], axis=1)
        rb = jnp.concatenate([rb[:, :, 1:2], rb, rb[:, :, Ho - 2:Ho - 1]],
                             axis=2)
        return rb
    return r


def _conv_stack_kernel(x_ref,
                       w0, b0, w1, b1, w2, b2, w3, b3, w4, b4,
                       w5, b5, w6, b6, w7, b7, w8, b8,
                       o_ref, *scratch, BB):
    w_refs = (w0, w1, w2, w3, w4, w5, w6, w7, w8)
    b_refs = (b0, b1, b2, b3, b4, b5, b6, b7, b8)
    NH = 1
    BBh = BB // NH
    vals = [x_ref[pl.ds(h * BBh, BBh)] for h in range(NH)]
    for li in range(9):
        for h in range(NH):
            vals[h] = _layer(li, vals[h], w_refs[li], b_refs[li], BBh)
    for h in range(NH):
        o_ref[pl.ds(h * BBh, BBh)] = vals[h]


def _prep_weights(conv_ws):
    out = []
    for li, (H, cin, cout, pool) in enumerate(_L):
        w = conv_ws[li]
        wc = w.reshape(3, 3, cin, cout).transpose(1, 2, 0, 3)
        out.append(wc.reshape(3 * cin, 3 * cout))
    return out


def _conv_stack(xp, conv_ws, conv_bs, BB):
    n = xp.shape[0]
    BBh = BB // 2
    in_specs = [pl.BlockSpec((BB, 34, 40, 128), lambda i: (i, 0, 0, 0))]
    operands = [xp]
    for w, b in zip(conv_ws, conv_bs):
        in_specs.append(pl.BlockSpec(w.shape, lambda i: (0, 0)))
        in_specs.append(pl.BlockSpec(b.shape, lambda i: (0, 0)))
        operands.append(w)
        operands.append(b)
    return pl.pallas_call(
        functools.partial(_conv_stack_kernel, BB=BB),
        out_shape=jax.ShapeDtypeStruct((n, 4, 4, 512), jnp.bfloat16),
        grid_spec=pltpu.PrefetchScalarGridSpec(
            num_scalar_prefetch=0,
            grid=(n // BB,),
            in_specs=in_specs,
            out_specs=pl.BlockSpec((BB, 4, 4, 512), lambda i: (i, 0, 0, 0)),
        ),
        compiler_params=pltpu.CompilerParams(
            dimension_semantics=("parallel",),
            vmem_limit_bytes=64 * 1024 * 1024),
    )(*operands)


def _fc_kernel(a_ref, w_ref, b_ref, o_ref, *, relu):
    r = jnp.dot(a_ref[...], w_ref[...], preferred_element_type=jnp.float32)
    r = r + b_ref[...]
    if relu:
        r = jnp.maximum(r, 0.0)
    o_ref[...] = r.astype(o_ref.dtype)


def _fc(a, w_packed, b, *, relu, out_dtype):
    m, k = a.shape
    n_blocks, kw, tn = w_packed.shape
    n = n_blocks * tn
    return pl.pallas_call(
        functools.partial(_fc_kernel, relu=relu),
        out_shape=jax.ShapeDtypeStruct((m, n), out_dtype),
        grid_spec=pltpu.PrefetchScalarGridSpec(
            num_scalar_prefetch=0,
            grid=(n_blocks,),
            in_specs=[
                pl.BlockSpec((m, k), lambda j: (0, 0)),
                pl.BlockSpec((None, k, tn), lambda j: (j, 0, 0)),
                pl.BlockSpec((1, tn), lambda j: (0, j)),
            ],
            out_specs=pl.BlockSpec((m, tn), lambda j: (0, j)),
        ),
        compiler_params=pltpu.CompilerParams(
            dimension_semantics=("parallel",),
            vmem_limit_bytes=48 * 1024 * 1024),
    )(a, w_packed, b)


def kernel(x, conv0_w, conv0_b, conv1_w, conv1_b, conv2_w, conv2_b,
           conv3_w, conv3_b, conv4_w, conv4_b, conv5_w, conv5_b,
           conv6_w, conv6_b, conv7_w, conv7_b, conv8_w, conv8_b,
           fc1_w, fc1_b, fc2_w, fc2_b, fc3_w, fc3_b):
    x_nhwc = jnp.transpose(x, (0, 2, 3, 1)).astype(jnp.bfloat16)
    xp = jnp.pad(x_nhwc, ((0, 0), (1, 1), (1, 1), (0, 0)), mode="reflect")
    xp = jnp.pad(xp, ((0, 0), (0, 0), (0, 6), (0, 125)))
    conv_ws = _prep_weights((conv0_w, conv1_w, conv2_w, conv3_w, conv4_w,
                             conv5_w, conv6_w, conv7_w, conv8_w))
    conv_bs = (conv0_b, conv1_b, conv2_b, conv3_b, conv4_b,
               conv5_b, conv6_b, conv7_b, conv8_b)
    feat = _conv_stack(xp, conv_ws, conv_bs, BB=8)
    a = feat.reshape(x.shape[0], 8192)
    a = _fc(a, fc1_w, fc1_b, relu=True, out_dtype=jnp.bfloat16)
    a = _fc(a, fc2_w, fc2_b, relu=True, out_dtype=jnp.bfloat16)
    logits = _fc(a, fc3_w, fc3_b, relu=False, out_dtype=jnp.float32)
    return logits[:, :100]


# speedup vs baseline: 1.0057x; 1.0057x over previous
import functools

import jax
import jax.numpy as jnp
from jax.experimental import pallas as pl
from jax.experimental.pallas import tpu as pltpu

_L = (
    (32, 3, 64, False),
    (32, 64, 64, True),
    (16, 64, 128, False),
    (16, 128, 128, True),
    (8, 128, 256, False),
    (8, 256, 256, False),
    (8, 256, 256, False),
    (8, 256, 256, True),
    (4, 256, 512, False),
)


def _fill_borders(P, H, W):
    P[:, pl.ds(0, 1), pl.ds(1, W), :] = P[:, pl.ds(2, 1), pl.ds(1, W), :]
    P[:, pl.ds(H + 1, 1), pl.ds(1, W), :] = P[:, pl.ds(H - 1, 1), pl.ds(1, W), :]
    P[:, :, pl.ds(0, 1), :] = P[:, :, pl.ds(2, 1), :]
    P[:, :, pl.ds(W + 1, 1), :] = P[:, :, pl.ds(W - 1, 1), :]


def _layer(li, P, w_ref, b_ref, BBh):
    H, cin, cout, pool = _L[li]
    W = H
    R = BBh * (H + 2) * W
    patch = jnp.concatenate(
        [P[:, :, dx:dx + W, 0:cin].reshape(R, cin) for dx in range(3)],
        axis=1)
    y = jnp.dot(patch, w_ref[...], preferred_element_type=jnp.float32)
    y = y.reshape(BBh, H + 2, W, 3 * cout)
    acc = (y[:, 0:H, :, 0:cout]
           + y[:, 1:H + 1, :, cout:2 * cout]
           + y[:, 2:H + 2, :, 2 * cout:3 * cout])
    if pool:
        Ho = H // 2
        a5 = acc.reshape(BBh, Ho, 2, W, cout)
        a = jnp.maximum(a5[:, :, 0], a5[:, :, 1])
        b5 = a.reshape(BBh, Ho, Ho, 2, cout)
        acc = jnp.maximum(b5[:, :, :, 0], b5[:, :, :, 1])
    else:
        Ho = H
    r = jnp.maximum(acc + b_ref[...], 0.0)
    r = r.astype(jnp.bfloat16)
    if li < 8:
        rb = jnp.concatenate([r[:, 1:2], r, r[:, Ho - 2:Ho - 1]], axis=1)
        rb = jnp.concatenate([rb[:, :, 1:2], rb, rb[:, :, Ho - 2:Ho - 1]],
                             axis=2)
        return rb
    return r


def _conv_stack_kernel(x_ref,
                       w0, b0, w1, b1, w2, b2, w3, b3, w4, b4,
                       w5, b5, w6, b6, w7, b7, w8, b8,
                       o_ref, *scratch, BB):
    w_refs = (w0, w1, w2, w3, w4, w5, w6, w7, w8)
    b_refs = (b0, b1, b2, b3, b4, b5, b6, b7, b8)
    NH = 4
    BBh = BB // NH
    vals = [x_ref[pl.ds(h * BBh, BBh)] for h in range(NH)]
    for li in range(9):
        for h in range(NH):
            vals[h] = _layer(li, vals[h], w_refs[li], b_refs[li], BBh)
    for h in range(NH):
        o_ref[pl.ds(h * BBh, BBh)] = vals[h]


def _prep_weights(conv_ws):
    out = []
    for li, (H, cin, cout, pool) in enumerate(_L):
        w = conv_ws[li]
        wc = w.reshape(3, 3, cin, cout).transpose(1, 2, 0, 3)
        out.append(wc.reshape(3 * cin, 3 * cout))
    return out


def _conv_stack(xp, conv_ws, conv_bs, BB):
    n = xp.shape[0]
    BBh = BB // 2
    in_specs = [pl.BlockSpec((BB, 34, 40, 128), lambda i: (i, 0, 0, 0))]
    operands = [xp]
    for w, b in zip(conv_ws, conv_bs):
        in_specs.append(pl.BlockSpec(w.shape, lambda i: (0, 0)))
        in_specs.append(pl.BlockSpec(b.shape, lambda i: (0, 0)))
        operands.append(w)
        operands.append(b)
    return pl.pallas_call(
        functools.partial(_conv_stack_kernel, BB=BB),
        out_shape=jax.ShapeDtypeStruct((n, 4, 4, 512), jnp.bfloat16),
        grid_spec=pltpu.PrefetchScalarGridSpec(
            num_scalar_prefetch=0,
            grid=(n // BB,),
            in_specs=in_specs,
            out_specs=pl.BlockSpec((BB, 4, 4, 512), lambda i: (i, 0, 0, 0)),
        ),
        compiler_params=pltpu.CompilerParams(
            dimension_semantics=("parallel",),
            vmem_limit_bytes=64 * 1024 * 1024),
    )(*operands)


def _fc_kernel(a_ref, w_ref, b_ref, o_ref, *, relu):
    r = jnp.dot(a_ref[...], w_ref[...], preferred_element_type=jnp.float32)
    r = r + b_ref[...]
    if relu:
        r = jnp.maximum(r, 0.0)
    o_ref[...] = r.astype(o_ref.dtype)


def _fc(a, w_packed, b, *, relu, out_dtype):
    m, k = a.shape
    n_blocks, kw, tn = w_packed.shape
    n = n_blocks * tn
    return pl.pallas_call(
        functools.partial(_fc_kernel, relu=relu),
        out_shape=jax.ShapeDtypeStruct((m, n), out_dtype),
        grid_spec=pltpu.PrefetchScalarGridSpec(
            num_scalar_prefetch=0,
            grid=(n_blocks,),
            in_specs=[
                pl.BlockSpec((m, k), lambda j: (0, 0)),
                pl.BlockSpec((None, k, tn), lambda j: (j, 0, 0)),
                pl.BlockSpec((1, tn), lambda j: (0, j)),
            ],
            out_specs=pl.BlockSpec((m, tn), lambda j: (0, j)),
        ),
        compiler_params=pltpu.CompilerParams(
            dimension_semantics=("parallel",),
            vmem_limit_bytes=48 * 1024 * 1024),
    )(a, w_packed, b)


def kernel(x, conv0_w, conv0_b, conv1_w, conv1_b, conv2_w, conv2_b,
           conv3_w, conv3_b, conv4_w, conv4_b, conv5_w, conv5_b,
           conv6_w, conv6_b, conv7_w, conv7_b, conv8_w, conv8_b,
           fc1_w, fc1_b, fc2_w, fc2_b, fc3_w, fc3_b):
    x_nhwc = jnp.transpose(x, (0, 2, 3, 1)).astype(jnp.bfloat16)
    xp = jnp.pad(x_nhwc, ((0, 0), (1, 1), (1, 1), (0, 0)), mode="reflect")
    xp = jnp.pad(xp, ((0, 0), (0, 0), (0, 6), (0, 125)))
    conv_ws = _prep_weights((conv0_w, conv1_w, conv2_w, conv3_w, conv4_w,
                             conv5_w, conv6_w, conv7_w, conv8_w))
    conv_bs = (conv0_b, conv1_b, conv2_b, conv3_b, conv4_b,
               conv5_b, conv6_b, conv7_b, conv8_b)
    feat = _conv_stack(xp, conv_ws, conv_bs, BB=8)
    a = feat.reshape(x.shape[0], 8192)
    a = _fc(a, fc1_w, fc1_b, relu=True, out_dtype=jnp.bfloat16)
    a = _fc(a, fc2_w, fc2_b, relu=True, out_dtype=jnp.bfloat16)
    logits = _fc(a, fc3_w, fc3_b, relu=False, out_dtype=jnp.float32)
    return logits[:, :100]


# speedup vs baseline: 1.0510x; 1.0451x over previous
import functools

import jax
import jax.numpy as jnp
from jax.experimental import pallas as pl
from jax.experimental.pallas import tpu as pltpu

_L = (
    (32, 3, 64, False),
    (32, 64, 64, True),
    (16, 64, 128, False),
    (16, 128, 128, True),
    (8, 128, 256, False),
    (8, 256, 256, False),
    (8, 256, 256, False),
    (8, 256, 256, True),
    (4, 256, 512, False),
)


def _layer(li, P, w_ref, b_ref, BBh):
    H, cin, cout, pool = _L[li]
    W = H
    R = BBh * (H + 2) * W
    patch = jnp.concatenate(
        [P[:, :, dx:dx + W, 0:cin].reshape(R, cin) for dx in range(3)],
        axis=1)
    y = jnp.dot(patch, w_ref[...], preferred_element_type=jnp.float32)
    y = y.reshape(BBh, H + 2, W, 3 * cout)
    acc = (y[:, 0:H, :, 0:cout]
           + y[:, 1:H + 1, :, cout:2 * cout]
           + y[:, 2:H + 2, :, 2 * cout:3 * cout])
    if pool:
        Ho = H // 2
        a5 = acc.reshape(BBh, Ho, 2, W, cout)
        a = jnp.maximum(a5[:, :, 0], a5[:, :, 1])
        b5 = a.reshape(BBh, Ho, Ho, 2, cout)
        acc = jnp.maximum(b5[:, :, :, 0], b5[:, :, :, 1])
    else:
        Ho = H
    r = jnp.maximum(acc + b_ref[...], 0.0)
    r = r.astype(jnp.bfloat16)
    if li < 8:
        rb = jnp.concatenate([r[:, 1:2], r, r[:, Ho - 2:Ho - 1]], axis=1)
        rb = jnp.concatenate([rb[:, :, 1:2], rb, rb[:, :, Ho - 2:Ho - 1]],
                             axis=2)
        return rb
    return r


def _conv_stack_kernel(x_ref,
                       w0, b0, w1, b1, w2, b2, w3, b3, w4, b4,
                       w5, b5, w6, b6, w7, b7, w8, b8,
                       o_ref, *, BB):
    w_refs = (w0, w1, w2, w3, w4, w5, w6, w7, w8)
    b_refs = (b0, b1, b2, b3, b4, b5, b6, b7, b8)
    BBh = BB // 2
    vals = [x_ref[pl.ds(h * BBh, BBh)] for h in range(2)]
    for li in range(9):
        for h in range(2):
            vals[h] = _layer(li, vals[h], w_refs[li], b_refs[li], BBh)
    for h in range(2):
        o_ref[pl.ds(h * BBh, BBh)] = vals[h]


def _prep_weights(conv_ws):
    out = []
    for li, (H, cin, cout, pool) in enumerate(_L):
        w = conv_ws[li]
        wc = w.reshape(3, 3, cin, cout).transpose(1, 2, 0, 3)
        out.append(wc.reshape(3 * cin, 3 * cout))
    return out


def _conv_stack(xp, conv_ws, conv_bs, BB):
    n = xp.shape[0]
    BBh = BB // 2
    in_specs = [pl.BlockSpec((BB, 34, 40, 128), lambda i: (i, 0, 0, 0))]
    operands = [xp]
    for w, b in zip(conv_ws, conv_bs):
        in_specs.append(pl.BlockSpec(w.shape, lambda i: (0, 0)))
        in_specs.append(pl.BlockSpec(b.shape, lambda i: (0, 0)))
        operands.append(w)
        operands.append(b)
    return pl.pallas_call(
        functools.partial(_conv_stack_kernel, BB=BB),
        out_shape=jax.ShapeDtypeStruct((n, 4, 4, 512), jnp.bfloat16),
        grid_spec=pltpu.PrefetchScalarGridSpec(
            num_scalar_prefetch=0,
            grid=(n // BB,),
            in_specs=in_specs,
            out_specs=pl.BlockSpec((BB, 4, 4, 512), lambda i: (i, 0, 0, 0)),
        ),
        compiler_params=pltpu.CompilerParams(
            dimension_semantics=("parallel",),
            vmem_limit_bytes=64 * 1024 * 1024),
    )(*operands)


def _fc_kernel(a_ref, w_ref, b_ref, o_ref, *, relu):
    r = jnp.dot(a_ref[...], w_ref[...], preferred_element_type=jnp.float32)
    r = r + b_ref[...]
    if relu:
        r = jnp.maximum(r, 0.0)
    o_ref[...] = r.astype(o_ref.dtype)


def _fc(a, w_packed, b, *, relu, out_dtype):
    m, k = a.shape
    n_blocks, kw, tn = w_packed.shape
    n = n_blocks * tn
    return pl.pallas_call(
        functools.partial(_fc_kernel, relu=relu),
        out_shape=jax.ShapeDtypeStruct((m, n), out_dtype),
        grid_spec=pltpu.PrefetchScalarGridSpec(
            num_scalar_prefetch=0,
            grid=(n_blocks,),
            in_specs=[
                pl.BlockSpec((m, k), lambda j: (0, 0)),
                pl.BlockSpec((None, k, tn), lambda j: (j, 0, 0)),
                pl.BlockSpec((1, tn), lambda j: (0, j)),
            ],
            out_specs=pl.BlockSpec((m, tn), lambda j: (0, j)),
        ),
        compiler_params=pltpu.CompilerParams(
            dimension_semantics=("parallel",),
            vmem_limit_bytes=48 * 1024 * 1024),
    )(a, w_packed, b)


def kernel(x, conv0_w, conv0_b, conv1_w, conv1_b, conv2_w, conv2_b,
           conv3_w, conv3_b, conv4_w, conv4_b, conv5_w, conv5_b,
           conv6_w, conv6_b, conv7_w, conv7_b, conv8_w, conv8_b,
           fc1_w, fc1_b, fc2_w, fc2_b, fc3_w, fc3_b):
    x_nhwc = jnp.transpose(x, (0, 2, 3, 1)).astype(jnp.bfloat16)
    xp = jnp.pad(x_nhwc, ((0, 0), (1, 1), (1, 1), (0, 0)), mode="reflect")
    xp = jnp.pad(xp, ((0, 0), (0, 0), (0, 6), (0, 125)))
    conv_ws = _prep_weights((conv0_w, conv1_w, conv2_w, conv3_w, conv4_w,
                             conv5_w, conv6_w, conv7_w, conv8_w))
    conv_bs = (conv0_b, conv1_b, conv2_b, conv3_b, conv4_b,
               conv5_b, conv6_b, conv7_b, conv8_b)
    feat = _conv_stack(xp, conv_ws, conv_bs, BB=8)
    a = feat.reshape(x.shape[0], 8192)
    a = _fc(a, fc1_w, fc1_b, relu=True, out_dtype=jnp.bfloat16)
    a = _fc(a, fc2_w, fc2_b, relu=True, out_dtype=jnp.bfloat16)
    logits = _fc(a, fc3_w, fc3_b, relu=False, out_dtype=jnp.float32)
    return logits[:, :100]
